# baseline (speedup 1.0000x reference)
import functools

import jax
import jax.numpy as jnp
from jax import lax
from jax.experimental import pallas as pl
from jax.experimental.pallas import tpu as pltpu
from jax.experimental.pallas import tpu_sc as plsc

NUM_EMBEDDINGS = 1000000
EMBED_DIM = 32
BATCH = 16384
HIST = 50

TOTAL = BATCH * HIST
NW = 32
PER_W = TOTAL // NW
CH = 1280
NGRP = PER_W // CH


def _build_kernel():
    mesh = plsc.VectorSubcoreMesh(core_axis_name="c", subcore_axis_name="s")

    @functools.partial(
        pl.kernel,
        mesh=mesh,
        out_type=jax.ShapeDtypeStruct((TOTAL, EMBED_DIM), jnp.float32),
        scratch_types=[
            pltpu.VMEM((PER_W,), jnp.int32),
            pltpu.VMEM((2, CH, EMBED_DIM), jnp.float32),
            pltpu.SemaphoreType.DMA,
            pltpu.SemaphoreType.DMA,
        ],
        compiler_params=pltpu.CompilerParams(use_tc_tiling_on_sc=False),
    )
    def gather_kernel(idx_hbm, table_hbm, out_hbm, idx_v, rows_v, gsem, osem):
        wid = lax.axis_index("s") * 2 + lax.axis_index("c")
        base = wid * PER_W
        pltpu.sync_copy(idx_hbm.at[pl.ds(base, PER_W)], idx_v)

        def fire(g, b):
            pltpu.async_copy(
                table_hbm.at[idx_v.at[pl.ds(g * CH, CH)]], rows_v.at[b], gsem
            )

        def drain_gather(b):
            pltpu.make_async_copy(
                table_hbm.at[idx_v.at[pl.ds(0, CH)]], rows_v.at[b], gsem
            ).wait()

        def wait_outcopy(b, g):
            pltpu.make_async_copy(
                rows_v.at[b], out_hbm.at[pl.ds(base + g * CH, CH)], osem
            ).wait()

        fire(0, 0)

        def step(g, carry):
            b_cur = lax.rem(g, 2)
            b_nxt = 1 - b_cur

            @pl.when(g > 0)
            def _():
                wait_outcopy(b_nxt, g - 1)

            fire(g + 1, b_nxt)
            drain_gather(b_cur)
            pltpu.async_copy(
                rows_v.at[b_cur], out_hbm.at[pl.ds(base + g * CH, CH)], osem
            )
            return carry

        lax.fori_loop(0, NGRP - 1, step, 0)

        b_last = (NGRP - 1) % 2
        wait_outcopy(1 - b_last, NGRP - 2)
        drain_gather(b_last)
        pltpu.sync_copy(
            rows_v.at[b_last], out_hbm.at[pl.ds(base + (NGRP - 1) * CH, CH)]
        )

    return gather_kernel


_gather = _build_kernel()


@jax.jit
def kernel(inputs, embedding):
    idx = inputs.astype(jnp.int32).reshape(TOTAL)
    out = _gather(idx, embedding)
    return out.reshape(BATCH, HIST, EMBED_DIM)


# speedup vs baseline: 1.0004x; 1.0004x over previous
import functools

import jax
import jax.numpy as jnp
from jax import lax
from jax.experimental import pallas as pl
from jax.experimental.pallas import tpu as pltpu
from jax.experimental.pallas import tpu_sc as plsc

NUM_EMBEDDINGS = 1000000
EMBED_DIM = 32
BATCH = 16384
HIST = 50

TOTAL = BATCH * HIST
NW = 32
PER_W = TOTAL // NW
GL = 64
KK = 20
CH = GL * KK
NGRP = PER_W // CH


def _build_kernel():
    mesh = plsc.VectorSubcoreMesh(core_axis_name="c", subcore_axis_name="s")

    @functools.partial(
        pl.kernel,
        mesh=mesh,
        out_type=jax.ShapeDtypeStruct((TOTAL, EMBED_DIM), jnp.float32),
        scratch_types=[
            pltpu.VMEM((PER_W,), jnp.int32),
            pltpu.VMEM((2, CH, EMBED_DIM), jnp.float32),
            pltpu.SemaphoreType.DMA,
            pltpu.SemaphoreType.DMA,
        ],
        compiler_params=pltpu.CompilerParams(use_tc_tiling_on_sc=False),
    )
    def gather_kernel(idx_hbm, table_hbm, out_hbm, idx_v, rows_v, gsem, osem):
        wid = lax.axis_index("s") * 2 + lax.axis_index("c")
        base = wid * PER_W
        pltpu.sync_copy(idx_hbm.at[pl.ds(base, PER_W)], idx_v)

        def fire(g, b):
            for j in range(KK):
                pltpu.async_copy(
                    table_hbm.at[idx_v.at[pl.ds(g * CH + j * GL, GL)]],
                    rows_v.at[b].at[pl.ds(j * GL, GL)],
                    gsem,
                )

        def drain_gather(b):
            pltpu.make_async_copy(
                table_hbm.at[idx_v.at[pl.ds(0, CH)]], rows_v.at[b], gsem
            ).wait()

        def wait_outcopy(b, g):
            pltpu.make_async_copy(
                rows_v.at[b], out_hbm.at[pl.ds(base + g * CH, CH)], osem
            ).wait()

        fire(0, 0)

        def step(g, carry):
            b_cur = lax.rem(g, 2)
            b_nxt = 1 - b_cur

            @pl.when(g > 0)
            def _():
                wait_outcopy(b_nxt, g - 1)

            fire(g + 1, b_nxt)
            drain_gather(b_cur)
            pltpu.async_copy(
                rows_v.at[b_cur], out_hbm.at[pl.ds(base + g * CH, CH)], osem
            )
            return carry

        lax.fori_loop(0, NGRP - 1, step, 0)

        b_last = (NGRP - 1) % 2
        wait_outcopy(1 - b_last, NGRP - 2)
        drain_gather(b_last)
        pltpu.sync_copy(
            rows_v.at[b_last], out_hbm.at[pl.ds(base + (NGRP - 1) * CH, CH)]
        )

    return gather_kernel


_gather = _build_kernel()


@jax.jit
def kernel(inputs, embedding):
    idx = inputs.astype(jnp.int32).reshape(TOTAL)
    out = _gather(idx, embedding)
    return out.reshape(BATCH, HIST, EMBED_DIM)


# speedup vs baseline: 1.1766x; 1.1761x over previous
import functools

import jax
import jax.numpy as jnp
from jax import lax
from jax.experimental import pallas as pl
from jax.experimental.pallas import tpu as pltpu
from jax.experimental.pallas import tpu_sc as plsc

NUM_EMBEDDINGS = 1000000
EMBED_DIM = 32
BATCH = 16384
HIST = 50

LANE = 128
TOTAL = BATCH * HIST
NROWS = TOTAL // LANE
NW = 32
ROWS_PER_W = NROWS // NW
KK = 8
NGRP = ROWS_PER_W // KK
NBUF = 3


def _build_kernel():
    mesh = plsc.VectorSubcoreMesh(core_axis_name="c", subcore_axis_name="s")

    @functools.partial(
        pl.kernel,
        mesh=mesh,
        out_type=jax.ShapeDtypeStruct((NROWS, LANE, EMBED_DIM), jnp.float32),
        scratch_types=[
            pltpu.VMEM((ROWS_PER_W, LANE), jnp.int32),
            pltpu.VMEM((NBUF, KK, LANE, EMBED_DIM), jnp.float32),
            pltpu.SemaphoreType.DMA,
            pltpu.SemaphoreType.DMA,
        ],
        compiler_params=pltpu.CompilerParams(use_tc_tiling_on_sc=False),
    )
    def gather_kernel(idx_hbm, table_hbm, out_hbm, idx_v, rows_v, gsem, osem):
        wid = lax.axis_index("s") * 2 + lax.axis_index("c")
        base = wid * ROWS_PER_W
        pltpu.sync_copy(idx_hbm.at[pl.ds(base, ROWS_PER_W)], idx_v)

        def fire(g, b):
            for j in range(KK):
                pltpu.async_copy(
                    table_hbm.at[idx_v.at[g * KK + j]], rows_v.at[b, j], gsem
                )

        def drain_gathers(b):
            for j in range(KK):
                pltpu.make_async_copy(
                    table_hbm.at[idx_v.at[j]], rows_v.at[b, j], gsem
                ).wait()

        def wait_outcopy(b, g):
            pltpu.make_async_copy(
                rows_v.at[b], out_hbm.at[pl.ds(base + g * KK, KK)], osem
            ).wait()

        for p in range(NBUF - 1):
            fire(p, p)

        def step(g, carry):
            b = lax.rem(g, NBUF)
            drain_gathers(b)

            @pl.when(g > 0)
            def _():
                wait_outcopy(lax.rem(g + NBUF - 1, NBUF), g - 1)

            @pl.when(g < NGRP - (NBUF - 1))
            def _():
                fire(g + NBUF - 1, lax.rem(g + NBUF - 1, NBUF))

            pltpu.async_copy(
                rows_v.at[b], out_hbm.at[pl.ds(base + g * KK, KK)], osem
            )
            return carry

        lax.fori_loop(0, NGRP, step, 0)
        wait_outcopy((NGRP - 1) % NBUF, NGRP - 1)

    return gather_kernel


_gather = _build_kernel()


@jax.jit
def kernel(inputs, embedding):
    idx = inputs.astype(jnp.int32).reshape(NROWS, LANE)
    out = _gather(idx, embedding)
    return out.reshape(BATCH, HIST, EMBED_DIM)
